# baseline (speedup 1.0000x reference)
import functools

import jax
import jax.numpy as jnp
from jax import lax
from jax.experimental import pallas as pl
from jax.experimental.pallas import tpu as pltpu
from jax.experimental.pallas import tpu_sc as plsc

D_MODEL = 64
N_TF = 12
TPAD = 16
N_T = 16384
N_R = 200
LANES = 16

_DNUMS = lax.GatherDimensionNumbers(
    offset_dims=(), collapsed_slice_dims=(0,), start_index_map=(0,))


def _make_kernel():
    info = plsc.get_sparse_core_info()
    nw = info.num_cores * info.num_subcores
    t_per_w = N_T // nw
    n_tb = t_per_w // LANES
    NPRE = 4

    mesh = plsc.VectorSubcoreMesh(core_axis_name="c", subcore_axis_name="s")

    @functools.partial(
        pl.kernel,
        mesh=mesh,
        compiler_params=pltpu.CompilerParams(needs_layout_passes=False),
        out_type=jax.ShapeDtypeStruct((N_R, D_MODEL, N_T), jnp.float32),
        scratch_types=[
            pltpu.VMEM((TPAD * D_MODEL,), jnp.float32),
            pltpu.VMEM((NPRE, t_per_w), jnp.int32),
            pltpu.VMEM((2, D_MODEL, t_per_w), jnp.float32),
            pltpu.SemaphoreType.DMA((NPRE,)),
            pltpu.SemaphoreType.DMA((2,)),
        ],
    )
    def emb_kernel(tab_hbm, idx_hbm, out_hbm, tab_v, idx_v, blk_v, isem, osem):
        cid = lax.axis_index("c")
        sid = lax.axis_index("s")
        wid = sid * info.num_cores + cid
        t0 = wid * t_per_w

        pltpu.sync_copy(tab_hbm, tab_v)

        def fire_idx(r, slot):
            return pltpu.async_copy(
                idx_hbm.at[r, pl.ds(t0, t_per_w)], idx_v.at[slot],
                isem.at[slot])

        def wait_idx(r, slot):
            pltpu.make_async_copy(
                idx_hbm.at[r, pl.ds(t0, t_per_w)], idx_v.at[slot],
                isem.at[slot]).wait()

        def fire_out(r, slot):
            return pltpu.async_copy(
                blk_v.at[slot], out_hbm.at[r, :, pl.ds(t0, t_per_w)],
                osem.at[slot])

        def wait_out(r, slot):
            pltpu.make_async_copy(
                blk_v.at[slot], out_hbm.at[r, :, pl.ds(t0, t_per_w)],
                osem.at[slot]).wait()

        for p in range(NPRE):
            fire_idx(p, p)

        def outer(g, carry):
            for par in range(NPRE):
                r = g * NPRE + par
                bslot = par % 2
                wait_idx(r, par)

                @pl.when(r >= 2)
                def _():
                    wait_out(r - 2, bslot)

                def cbody(c, c2):
                    tabvec = tab_v[pl.ds(c * TPAD, TPAD)]

                    @plsc.parallel_loop(0, n_tb, unroll=16)
                    def _(tb):
                        idxv = idx_v[par, pl.ds(tb * LANES, LANES)]
                        vals = lax.gather(
                            tabvec, idxv.reshape(LANES, 1), _DNUMS, (1,),
                            mode=lax.GatherScatterMode.PROMISE_IN_BOUNDS)
                        blk_v[bslot, c, pl.ds(tb * LANES, LANES)] = vals

                    return c2

                lax.fori_loop(0, D_MODEL, cbody, 0)
                fire_out(r, bslot)

                @pl.when(r + NPRE < N_R)
                def _():
                    fire_idx(r + NPRE, par)
            return carry

        lax.fori_loop(0, N_R // NPRE, outer, 0)

        wait_out(N_R - 2, 0)
        wait_out(N_R - 1, 1)

    return emb_kernel


_EMB_KERNEL = _make_kernel()


def kernel(tf_idx, emb_weight):
    tab_t = jnp.zeros((D_MODEL, TPAD), jnp.float32)
    tab_t = tab_t.at[:, :N_TF].set(emb_weight.T).reshape(TPAD * D_MODEL)
    idx_t = tf_idx.T
    out_t = _EMB_KERNEL(tab_t, idx_t)
    return jnp.transpose(out_t, (2, 0, 1))
